# baseline (speedup 1.0000x reference)
import functools

import jax
import jax.numpy as jnp
from jax import lax
from jax.experimental import pallas as pl
from jax.experimental.pallas import tpu as pltpu
from jax.experimental.pallas import tpu_sc as plsc

BATCH = 16384
SEQ = 50
DIM = 64
VOCAB = 1_000_000
NUM_CORES = 2
NUM_SUBCORES = 16
D_PER_CORE = DIM // NUM_CORES
B_PER_SUB = BATCH // NUM_SUBCORES
SLAB = 4
N_SLABS = 12
ROW_CHUNK = 62464
ROW_ALIGNED = 999936

_mesh = plsc.VectorSubcoreMesh(core_axis_name="c", subcore_axis_name="s")


@functools.partial(
    pl.kernel,
    mesh=_mesh,
    out_type=jax.ShapeDtypeStruct((SEQ, DIM, BATCH), jnp.float32),
    scratch_types=[
        pltpu.VMEM_SHARED((VOCAB,), jnp.float32),
        pltpu.VMEM((SEQ * B_PER_SUB,), jnp.int32),
        pltpu.VMEM((2, SLAB, 1, B_PER_SUB), jnp.float32),
        pltpu.VMEM((128,), jnp.float32),
        pltpu.SemaphoreType.DMA,
        pltpu.SemaphoreType.DMA,
        pltpu.SemaphoreType.DMA,
    ],
)
def _gather_kernel(
    table_hbm, tails_hbm, idx_hbm, out_hbm, row_sp, idx_v, gbuf, tail_buf,
    gsem, s0, s1
):
    c = lax.axis_index("c")
    t = lax.axis_index("s")
    b0 = t * B_PER_SUB
    ssems = (s0, s1)

    for si in range(SEQ):
        pltpu.async_copy(
            idx_hbm.at[si].at[pl.ds(b0, B_PER_SUB)],
            idx_v.at[pl.ds(si * B_PER_SUB, B_PER_SUB)],
            gsem,
        )
    for si in range(SEQ):
        pltpu.make_async_copy(
            idx_hbm.at[0].at[pl.ds(0, B_PER_SUB)],
            idx_v.at[pl.ds(0, B_PER_SUB)],
            gsem,
        ).wait()


    def fire_gather(s, bsel, r):
        pltpu.async_copy(
            row_sp.at[idx_v.at[pl.ds(s * B_PER_SUB, B_PER_SUB)]],
            gbuf.at[bsel, r, 0],
            gsem,
        )

    def drain_gather():
        pltpu.make_async_copy(
            table_hbm.at[0].at[pl.ds(0, B_PER_SUB)], gbuf.at[0, 0, 0], gsem
        ).wait()

    def store_slab(so, bsel, d, n):
        pltpu.async_copy(
            gbuf.at[bsel, pl.ds(0, n)],
            out_hbm.at[pl.ds(so * SLAB, n), pl.ds(d, 1), pl.ds(b0, B_PER_SUB)],
            ssems[bsel],
        )

    def drain_store(bsel, n):
        pltpu.make_async_copy(
            gbuf.at[bsel, pl.ds(0, n)],
            out_hbm.at[pl.ds(0, n), pl.ds(0, 1), pl.ds(0, B_PER_SUB)],
            ssems[bsel],
        ).wait()

    def gather_slab(so, bsel, d):
        def inner(s4, carry3):
            fire_gather(so * SLAB + s4, bsel, s4)

            @pl.when(s4 > 0)
            def _lagged():
                drain_gather()

            return carry3

        lax.fori_loop(0, SLAB, inner, 0)
        drain_gather()
        store_slab(so, bsel, d, SLAB)

    def per_dim(dd, carry):
        d = c * D_PER_CORE + dd
        plsc.subcore_barrier()

        pltpu.sync_copy(
            table_hbm.at[d].at[pl.ds(t * ROW_CHUNK, ROW_CHUNK)],
            row_sp.at[pl.ds(t * ROW_CHUNK, ROW_CHUNK)],
        )

        @pl.when(t == 0)
        def _load_span():
            pltpu.sync_copy(
                table_hbm.at[d].at[pl.ds(NUM_SUBCORES * ROW_CHUNK, 512)],
                row_sp.at[pl.ds(NUM_SUBCORES * ROW_CHUNK, 512)],
            )

        @pl.when(t == 1)
        def _load_tail():
            pltpu.sync_copy(tails_hbm.at[d], tail_buf)
            pltpu.sync_copy(
                tail_buf.at[pl.ds(0, 64)], row_sp.at[pl.ds(ROW_ALIGNED, 64)]
            )

        plsc.subcore_barrier()

        gather_slab(0, 0, d)
        gather_slab(1, 1, d)

        def per_pair(p, carry2):
            so = 2 * p + 2
            drain_store(0, SLAB)
            gather_slab(so, 0, d)
            drain_store(1, SLAB)
            gather_slab(so + 1, 1, d)
            return carry2

        lax.fori_loop(0, (N_SLABS - 2) // 2, per_pair, 0)

        drain_store(0, SLAB)
        fire_gather(48, 0, 0)
        fire_gather(49, 0, 1)
        drain_gather()
        drain_gather()
        pltpu.sync_copy(
            gbuf.at[0, pl.ds(0, 2)],
            out_hbm.at[pl.ds(48, 2), pl.ds(d, 1), pl.ds(b0, B_PER_SUB)],
        )
        drain_store(1, SLAB)
        return carry

    lax.fori_loop(0, D_PER_CORE, per_dim, 0)


def kernel(x, weight):
    w_t = weight.T
    tails = jnp.pad(w_t[:, ROW_ALIGNED:], ((0, 0), (0, 128 - (VOCAB - ROW_ALIGNED))))
    out_t = _gather_kernel(w_t, tails, x.astype(jnp.int32).T)
    return out_t.transpose(2, 0, 1)


# speedup vs baseline: 1.0091x; 1.0091x over previous
import functools

import jax
import jax.numpy as jnp
from jax import lax
from jax.experimental import pallas as pl
from jax.experimental.pallas import tpu as pltpu
from jax.experimental.pallas import tpu_sc as plsc

BATCH = 16384
SEQ = 50
DIM = 64
VOCAB = 1_000_000
NUM_CORES = 2
NUM_SUBCORES = 16
D_PER_CORE = DIM // NUM_CORES
B_PER_SUB = BATCH // NUM_SUBCORES
SLAB = 8
N_SLABS = 6
ROW_CHUNK = 62464
ROW_ALIGNED = 999936

_mesh = plsc.VectorSubcoreMesh(core_axis_name="c", subcore_axis_name="s")


@functools.partial(
    pl.kernel,
    mesh=_mesh,
    out_type=jax.ShapeDtypeStruct((SEQ, DIM, BATCH), jnp.float32),
    scratch_types=[
        pltpu.VMEM_SHARED((VOCAB,), jnp.float32),
        pltpu.VMEM((SEQ * B_PER_SUB,), jnp.int32),
        pltpu.VMEM((2, SLAB, 1, B_PER_SUB), jnp.float32),
        pltpu.VMEM((128,), jnp.float32),
        pltpu.SemaphoreType.DMA,
        pltpu.SemaphoreType.DMA,
        pltpu.SemaphoreType.DMA,
    ],
)
def _gather_kernel(
    table_hbm, tails_hbm, idx_hbm, out_hbm, row_sp, idx_v, gbuf, tail_buf,
    gsem, s0, s1
):
    c = lax.axis_index("c")
    t = lax.axis_index("s")
    b0 = t * B_PER_SUB
    ssems = (s0, s1)

    for si in range(SEQ):
        pltpu.async_copy(
            idx_hbm.at[si].at[pl.ds(b0, B_PER_SUB)],
            idx_v.at[pl.ds(si * B_PER_SUB, B_PER_SUB)],
            gsem,
        )
    for si in range(SEQ):
        pltpu.make_async_copy(
            idx_hbm.at[0].at[pl.ds(0, B_PER_SUB)],
            idx_v.at[pl.ds(0, B_PER_SUB)],
            gsem,
        ).wait()


    def fire_gather(s, bsel, r):
        pltpu.async_copy(
            row_sp.at[idx_v.at[pl.ds(s * B_PER_SUB, B_PER_SUB)]],
            gbuf.at[bsel, r, 0],
            gsem,
        )

    def drain_gather():
        pltpu.make_async_copy(
            table_hbm.at[0].at[pl.ds(0, B_PER_SUB)], gbuf.at[0, 0, 0], gsem
        ).wait()

    def store_slab(so, bsel, d, n):
        pltpu.async_copy(
            gbuf.at[bsel, pl.ds(0, n)],
            out_hbm.at[pl.ds(so * SLAB, n), pl.ds(d, 1), pl.ds(b0, B_PER_SUB)],
            ssems[bsel],
        )

    def drain_store(bsel, n):
        pltpu.make_async_copy(
            gbuf.at[bsel, pl.ds(0, n)],
            out_hbm.at[pl.ds(0, n), pl.ds(0, 1), pl.ds(0, B_PER_SUB)],
            ssems[bsel],
        ).wait()

    def gather_slab(so, bsel, d):
        def inner(s4, carry3):
            fire_gather(so * SLAB + s4, bsel, s4)

            @pl.when(s4 > 1)
            def _lagged():
                drain_gather()

            return carry3

        lax.fori_loop(0, SLAB, inner, 0)
        drain_gather()
        drain_gather()
        store_slab(so, bsel, d, SLAB)

    def per_dim(dd, carry):
        d = c * D_PER_CORE + dd
        plsc.subcore_barrier()

        pltpu.sync_copy(
            table_hbm.at[d].at[pl.ds(t * ROW_CHUNK, ROW_CHUNK)],
            row_sp.at[pl.ds(t * ROW_CHUNK, ROW_CHUNK)],
        )

        @pl.when(t == 0)
        def _load_span():
            pltpu.sync_copy(
                table_hbm.at[d].at[pl.ds(NUM_SUBCORES * ROW_CHUNK, 512)],
                row_sp.at[pl.ds(NUM_SUBCORES * ROW_CHUNK, 512)],
            )

        @pl.when(t == 1)
        def _load_tail():
            pltpu.sync_copy(tails_hbm.at[d], tail_buf)
            pltpu.sync_copy(
                tail_buf.at[pl.ds(0, 64)], row_sp.at[pl.ds(ROW_ALIGNED, 64)]
            )

        plsc.subcore_barrier()

        gather_slab(0, 0, d)
        gather_slab(1, 1, d)

        def per_pair(p, carry2):
            so = 2 * p + 2
            drain_store(0, SLAB)
            gather_slab(so, 0, d)
            drain_store(1, SLAB)
            gather_slab(so + 1, 1, d)
            return carry2

        lax.fori_loop(0, (N_SLABS - 2) // 2, per_pair, 0)

        drain_store(0, SLAB)
        fire_gather(48, 0, 0)
        fire_gather(49, 0, 1)
        drain_gather()
        drain_gather()
        pltpu.sync_copy(
            gbuf.at[0, pl.ds(0, 2)],
            out_hbm.at[pl.ds(48, 2), pl.ds(d, 1), pl.ds(b0, B_PER_SUB)],
        )
        drain_store(1, SLAB)
        return carry

    lax.fori_loop(0, D_PER_CORE, per_dim, 0)


def kernel(x, weight):
    w_t = weight.T
    tails = jnp.pad(w_t[:, ROW_ALIGNED:], ((0, 0), (0, 128 - (VOCAB - ROW_ALIGNED))))
    out_t = _gather_kernel(w_t, tails, x.astype(jnp.int32).T)
    return out_t.transpose(2, 0, 1)
